# baseline (speedup 1.0000x reference)
import functools

import jax
import jax.numpy as jnp
from jax import lax
from jax.experimental import pallas as pl
from jax.experimental.pallas import tpu as pltpu
from jax.experimental.pallas import tpu_sc as plsc

N_NODES = 10000
N_EDGES = 160000
HIDDEN = 256
HALF = 128
NUM_REL = 16

NC = 2
NS = 16
LANES = 16

EPT = N_EDGES // NS
CHUNK = 128
EPT_PAD = 10240
NCHUNK = EPT_PAD // CHUNK
ACC_ROWS = 10240
ZROWS = ACC_ROWS // NS
C_WORDS = ACC_ROWS * NUM_REL
C_ZWORDS = C_WORDS // NS
OUT_ROWS_PT = ACC_ROWS // NS

NBUF = 2
IBUF = 4
ZBUF = 2048


def _sc_aggregate(src2, dst_p, rel_p, nodes_split, z2d, z1d):
    mesh = plsc.VectorSubcoreMesh(core_axis_name="c", subcore_axis_name="s")

    @functools.partial(
        pl.kernel,
        out_type=(
            jax.ShapeDtypeStruct((2 * ACC_ROWS, HALF), jnp.float32),
            jax.ShapeDtypeStruct((N_NODES * NUM_REL,), jnp.float32),
            jax.ShapeDtypeStruct((N_NODES * NUM_REL,), jnp.float32),
        ),
        mesh=mesh,
        scratch_types=[
            [pltpu.VMEM((CHUNK, HALF), jnp.float32) for _ in range(NBUF)],
            [pltpu.VMEM((CHUNK,), jnp.int32) for _ in range(IBUF)],
            [pltpu.VMEM((CHUNK,), jnp.int32) for _ in range(IBUF)],
            [pltpu.VMEM((CHUNK,), jnp.int32) for _ in range(IBUF)],
            [pltpu.VMEM((CHUNK,), jnp.int32) for _ in range(NBUF)],
            pltpu.VMEM((CHUNK,), jnp.float32),
            pltpu.VMEM((ZBUF,), jnp.float32),
            pltpu.VMEM_SHARED((ACC_ROWS, HALF), jnp.float32),
            pltpu.VMEM_SHARED((C_WORDS,), jnp.float32),
            [pltpu.SemaphoreType.DMA for _ in range(NBUF)],
            [pltpu.SemaphoreType.DMA for _ in range(IBUF)],
            [pltpu.SemaphoreType.DMA for _ in range(NBUF)],
            pltpu.SemaphoreType.DMA,
        ],
    )
    def sc_kernel(src2_hbm, dst_hbm, rel_hbm, nodes_hbm, z2d_hbm, z1d_hbm,
                  out_nodes_hbm, out_c0_hbm, out_c1_hbm,
                  rows_v, srcs, dsts, rels, cidxr, ones_v, zbuf_v,
                  acc_s, c_s, gsem, isem, ssem, csem):
        cid = lax.axis_index("c")
        sid = lax.axis_index("s")

        pltpu.sync_copy(z2d_hbm, acc_s.at[pl.ds(sid * ZROWS, ZROWS)])
        pltpu.sync_copy(z1d_hbm, zbuf_v)
        for k in range(C_ZWORDS // ZBUF):
            pltpu.sync_copy(zbuf_v,
                            c_s.at[pl.ds(sid * C_ZWORDS + k * ZBUF, ZBUF)])
        for t in range(CHUNK // LANES):
            ones_v[pl.ds(t * LANES, LANES)] = jnp.full((LANES,), 1.0, jnp.float32)

        seg_base = sid * EPT_PAD
        src_base = cid * (NS * EPT_PAD) + seg_base

        def idx_start(j, q):
            pltpu.async_copy(src2_hbm.at[pl.ds(src_base + j * CHUNK, CHUNK)],
                             srcs[q], isem[q])
            pltpu.async_copy(dst_hbm.at[pl.ds(seg_base + j * CHUNK, CHUNK)],
                             dsts[q], isem[q])
            pltpu.async_copy(rel_hbm.at[pl.ds(seg_base + j * CHUNK, CHUNK)],
                             rels[q], isem[q])

        def idx_wait(q):
            for r in range(3):
                pltpu.make_async_copy(
                    dst_hbm.at[pl.ds(0, CHUNK)], dsts[q], isem[q]).wait()

        def gather_start(j, b, q):
            pltpu.async_copy(nodes_hbm.at[srcs[q]], rows_v[b], gsem[b])

        def gather_wait(b):
            pltpu.make_async_copy(nodes_hbm.at[srcs[0]], rows_v[b],
                                  gsem[b]).wait()

        def scatter_wait(b):
            pltpu.make_async_copy(z2d_hbm.at[pl.ds(0, CHUNK)], rows_v[b],
                                  ssem[b]).wait()

        def cscatter_wait():
            pltpu.make_async_copy(z1d_hbm.at[pl.ds(0, CHUNK)], ones_v,
                                  csem).wait()

        plsc.subcore_barrier()

        for q in range(IBUF):
            idx_start(q, q)
        idx_wait(0)
        gather_start(0, 0, 0)

        def step_body(s, carry):
            for u in range(IBUF):
                j = s * IBUF + u
                b = u % NBUF
                q = u
                bn = (u + 1) % NBUF
                qp = (u + 3) % IBUF

                @pl.when(j >= 1)
                def _():
                    scatter_wait(bn)

                    @pl.when(j + 3 < NCHUNK)
                    def _():
                        idx_start(j + 3, qp)

                @pl.when(j + 1 < NCHUNK)
                def _():
                    idx_wait((u + 1) % IBUF)
                    gather_start(j + 1, bn, (u + 1) % IBUF)

                gather_wait(b)

                @pl.when(cid == (u % 2))
                def _():
                    @pl.when(j >= 2)
                    def _():
                        cscatter_wait()

                    for t in range(CHUNK // LANES):
                        sl = pl.ds(t * LANES, LANES)
                        cidxr[u % 2][sl] = (dsts[q][sl] * NUM_REL
                                            + rels[q][sl])
                    pltpu.async_copy(ones_v, c_s.at[cidxr[u % 2]], csem,
                                     add=True)

                pltpu.async_copy(rows_v[b], acc_s.at[dsts[q]],
                                 ssem[b], add=True)
            return carry

        lax.fori_loop(0, NCHUNK // IBUF, step_body, 0)
        scatter_wait((NCHUNK - 1) % NBUF)
        cscatter_wait()
        plsc.subcore_barrier()

        pltpu.sync_copy(
            acc_s.at[pl.ds(sid * OUT_ROWS_PT, OUT_ROWS_PT)],
            out_nodes_hbm.at[pl.ds(cid * ACC_ROWS + sid * OUT_ROWS_PT,
                                   OUT_ROWS_PT)])

        cw = N_EDGES // NS
        for k in range(5):
            piece = pl.ds(0, cw // 5)
            csl = pl.ds(sid * cw + k * (cw // 5), cw // 5)
            pltpu.sync_copy(c_s.at[csl], zbuf_v.at[piece])

            @pl.when(cid == 0)
            def _():
                pltpu.sync_copy(zbuf_v.at[piece], out_c0_hbm.at[csl])

            @pl.when(cid == 1)
            def _():
                pltpu.sync_copy(zbuf_v.at[piece], out_c1_hbm.at[csl])

    return sc_kernel(src2, dst_p, rel_p, nodes_split, z2d, z1d)


BLK = 80
GRID = N_NODES // BLK
RIGHT_OFF = ACC_ROWS // BLK


def _tc_finish(x, node_part, C0, C1, Ws, bs2, Wm, bm2, rel_emb, gamma2, beta2):
    prec = lax.Precision.DEFAULT

    def body(x_ref, l_ref, r_ref, c0_ref, c1_ref, ws_ref, bs_ref, wm_ref,
             bm_ref, rel_ref, g_ref, b_ref, o_ref):
        xb = x_ref[...]
        cb = c0_ref[...] + c1_ref[...]
        deg = jnp.sum(cb, axis=1, keepdims=True)
        mrel = jnp.dot(rel_ref[...], wm_ref[...],
                       preferred_element_type=jnp.float32, precision=prec)
        t = (jnp.dot(l_ref[...], wm_ref[:HALF, :],
                     preferred_element_type=jnp.float32, precision=prec)
             + jnp.dot(r_ref[...], wm_ref[HALF:, :],
                       preferred_element_type=jnp.float32, precision=prec)
             + jnp.dot(cb, mrel, preferred_element_type=jnp.float32,
                       precision=prec))
        agg = (t + deg * bm_ref[...]) / jnp.maximum(deg, 1.0)
        u = (jnp.dot(xb, ws_ref[...],
                     preferred_element_type=jnp.float32, precision=prec)
             + bs_ref[...] + agg)
        g = 0.5 * u * (1.0 + lax.erf(u * 0.7071067811865476))
        r = g + xb
        mu = jnp.mean(r, axis=1, keepdims=True)
        var = jnp.mean((r - mu) ** 2, axis=1, keepdims=True)
        o_ref[...] = (r - mu) * lax.rsqrt(var + 1e-5) * g_ref[...] + b_ref[...]

    full = lambda shape: pl.BlockSpec(shape, lambda i: (0, 0))
    return pl.pallas_call(
        body,
        grid=(GRID,),
        in_specs=[
            pl.BlockSpec((BLK, HIDDEN), lambda i: (i, 0)),
            pl.BlockSpec((BLK, HALF), lambda i: (i, 0)),
            pl.BlockSpec((BLK, HALF), lambda i: (i + RIGHT_OFF, 0)),
            pl.BlockSpec((BLK, NUM_REL), lambda i: (i, 0)),
            pl.BlockSpec((BLK, NUM_REL), lambda i: (i, 0)),
            full((HIDDEN, HIDDEN)),
            full((1, HIDDEN)),
            full((HIDDEN, HIDDEN)),
            full((1, HIDDEN)),
            full((NUM_REL, HIDDEN)),
            full((1, HIDDEN)),
            full((1, HIDDEN)),
        ],
        out_specs=pl.BlockSpec((BLK, HIDDEN), lambda i: (i, 0)),
        out_shape=jax.ShapeDtypeStruct((N_NODES, HIDDEN), jnp.float32),
    )(x, node_part, node_part, C0, C1, Ws, bs2, Wm, bm2, rel_emb, gamma2,
      beta2)


def kernel(node_states, Ws, bs, Wm, bm, rel_emb, ln_gamma, ln_beta,
           edge_index, edge_type_ids):
    src = edge_index[0].astype(jnp.int32)
    dst = edge_index[1].astype(jnp.int32)
    rel = jnp.clip(edge_type_ids.astype(jnp.int32), 0, NUM_REL - 1)

    pad = EPT_PAD - EPT
    src_p = jnp.pad(src.reshape(NS, EPT), ((0, 0), (0, pad))).reshape(-1)
    dst_p = jnp.pad(dst.reshape(NS, EPT), ((0, 0), (0, pad)),
                    constant_values=N_NODES).reshape(-1)
    rel_p = jnp.pad(rel.reshape(NS, EPT), ((0, 0), (0, pad))).reshape(-1)
    src2 = jnp.concatenate([2 * src_p, 2 * src_p + 1])
    nodes_split = node_states.reshape(2 * N_NODES, HALF)
    z2d = jnp.zeros((ZROWS, HALF), jnp.float32)
    z1d = jnp.zeros((ZBUF,), jnp.float32)

    node_part, c0, c1 = _sc_aggregate(src2, dst_p, rel_p, nodes_split,
                                      z2d, z1d)
    C0 = c0.reshape(N_NODES, NUM_REL)
    C1 = c1.reshape(N_NODES, NUM_REL)

    return _tc_finish(node_states, node_part, C0, C1, Ws,
                      bs.reshape(1, HIDDEN), Wm, bm.reshape(1, HIDDEN),
                      rel_emb, ln_gamma.reshape(1, HIDDEN),
                      ln_beta.reshape(1, HIDDEN))


# speedup vs baseline: 1.1969x; 1.1969x over previous
import functools

import jax
import jax.numpy as jnp
from jax import lax
from jax.experimental import pallas as pl
from jax.experimental.pallas import tpu as pltpu
from jax.experimental.pallas import tpu_sc as plsc

N_NODES = 10000
N_EDGES = 160000
HIDDEN = 256
HALF = 128
NUM_REL = 16

NC = 2
NS = 16
LANES = 16

EPT = N_EDGES // NS
CHUNK = 128
EPT_PAD = 10240
NCHUNK = EPT_PAD // CHUNK
ACC_ROWS = 10240
ZROWS = ACC_ROWS // NS
C_WORDS = ACC_ROWS * NUM_REL
C_ZWORDS = C_WORDS // NS
OUT_ROWS_PT = ACC_ROWS // NS

NBUF = 2
IBUF = 4
ZBUF = 2048


def _sc_aggregate(src2, dst_p, rel_p, nodes_split, z2d, z1d):
    mesh = plsc.VectorSubcoreMesh(core_axis_name="c", subcore_axis_name="s")

    @functools.partial(
        pl.kernel,
        out_type=(
            jax.ShapeDtypeStruct((2 * N_NODES, HALF), jnp.float32),
            jax.ShapeDtypeStruct((N_NODES * NUM_REL,), jnp.float32),
            jax.ShapeDtypeStruct((N_NODES * NUM_REL,), jnp.float32),
        ),
        mesh=mesh,
        scratch_types=[
            [pltpu.VMEM((CHUNK, HALF), jnp.float32) for _ in range(NBUF)],
            [pltpu.VMEM((CHUNK,), jnp.int32) for _ in range(IBUF)],
            [pltpu.VMEM((CHUNK,), jnp.int32) for _ in range(IBUF)],
            [pltpu.VMEM((CHUNK,), jnp.int32) for _ in range(IBUF)],
            [pltpu.VMEM((CHUNK,), jnp.int32) for _ in range(NBUF)],
            pltpu.VMEM((CHUNK,), jnp.float32),
            pltpu.VMEM((ZBUF,), jnp.float32),
            pltpu.VMEM_SHARED((ACC_ROWS, HALF), jnp.float32),
            pltpu.VMEM_SHARED((C_WORDS,), jnp.float32),
            [pltpu.SemaphoreType.DMA for _ in range(NBUF)],
            [pltpu.SemaphoreType.DMA for _ in range(IBUF)],
            [pltpu.SemaphoreType.DMA for _ in range(NBUF)],
            pltpu.SemaphoreType.DMA,
        ],
    )
    def sc_kernel(src2_hbm, dst_hbm, rel_hbm, nodes_hbm, z2d_hbm, z1d_hbm,
                  out_nodes_hbm, out_c0_hbm, out_c1_hbm,
                  rows_v, srcs, dsts, rels, cidxr, ones_v, zbuf_v,
                  acc_s, c_s, gsem, isem, ssem, csem):
        cid = lax.axis_index("c")
        sid = lax.axis_index("s")

        pltpu.sync_copy(z2d_hbm, acc_s.at[pl.ds(sid * ZROWS, ZROWS)])
        pltpu.sync_copy(z1d_hbm, zbuf_v)
        for k in range(C_ZWORDS // ZBUF):
            pltpu.sync_copy(zbuf_v,
                            c_s.at[pl.ds(sid * C_ZWORDS + k * ZBUF, ZBUF)])
        for t in range(CHUNK // LANES):
            ones_v[pl.ds(t * LANES, LANES)] = jnp.full((LANES,), 1.0, jnp.float32)

        seg_base = sid * EPT_PAD
        src_base = cid * (NS * EPT_PAD) + seg_base

        def idx_start(j, q):
            pltpu.async_copy(src2_hbm.at[pl.ds(src_base + j * CHUNK, CHUNK)],
                             srcs[q], isem[q])
            pltpu.async_copy(dst_hbm.at[pl.ds(seg_base + j * CHUNK, CHUNK)],
                             dsts[q], isem[q])
            pltpu.async_copy(rel_hbm.at[pl.ds(seg_base + j * CHUNK, CHUNK)],
                             rels[q], isem[q])

        def idx_wait(q):
            for r in range(3):
                pltpu.make_async_copy(
                    dst_hbm.at[pl.ds(0, CHUNK)], dsts[q], isem[q]).wait()

        def gather_start(j, b, q):
            pltpu.async_copy(nodes_hbm.at[srcs[q]], rows_v[b], gsem[b])

        def gather_wait(b):
            pltpu.make_async_copy(nodes_hbm.at[srcs[0]], rows_v[b],
                                  gsem[b]).wait()

        def scatter_wait(b):
            pltpu.make_async_copy(z2d_hbm.at[pl.ds(0, CHUNK)], rows_v[b],
                                  ssem[b]).wait()

        def cscatter_wait():
            pltpu.make_async_copy(z1d_hbm.at[pl.ds(0, CHUNK)], ones_v,
                                  csem).wait()

        plsc.subcore_barrier()

        for q in range(IBUF):
            idx_start(q, q)
        idx_wait(0)
        gather_start(0, 0, 0)

        def step_body(s, carry):
            for u in range(IBUF):
                j = s * IBUF + u
                b = u % NBUF
                q = u
                bn = (u + 1) % NBUF
                qp = (u + 3) % IBUF

                @pl.when(j >= 1)
                def _():
                    scatter_wait(bn)

                    @pl.when(j + 3 < NCHUNK)
                    def _():
                        idx_start(j + 3, qp)

                @pl.when(j + 1 < NCHUNK)
                def _():
                    idx_wait((u + 1) % IBUF)
                    gather_start(j + 1, bn, (u + 1) % IBUF)

                gather_wait(b)

                @pl.when(cid == (u % 2))
                def _():
                    @pl.when(j >= 2)
                    def _():
                        cscatter_wait()

                    for t in range(CHUNK // LANES):
                        sl = pl.ds(t * LANES, LANES)
                        cidxr[u % 2][sl] = (dsts[q][sl] * NUM_REL
                                            + rels[q][sl])
                    pltpu.async_copy(ones_v, c_s.at[cidxr[u % 2]], csem,
                                     add=True)

                pltpu.async_copy(rows_v[b], acc_s.at[dsts[q]],
                                 ssem[b], add=True)
            return carry

        lax.fori_loop(0, NCHUNK // IBUF, step_body, 0)
        scatter_wait((NCHUNK - 1) % NBUF)
        cscatter_wait()
        plsc.subcore_barrier()

        @pl.when(sid < NS - 1)
        def _():
            pltpu.sync_copy(
                acc_s.at[pl.ds(sid * OUT_ROWS_PT, OUT_ROWS_PT)],
                out_nodes_hbm.at[pl.ds(cid * N_NODES + sid * OUT_ROWS_PT,
                                       OUT_ROWS_PT)])

        @pl.when(sid == NS - 1)
        def _():
            last = N_NODES - (NS - 1) * OUT_ROWS_PT
            pltpu.sync_copy(
                acc_s.at[pl.ds((NS - 1) * OUT_ROWS_PT, last)],
                out_nodes_hbm.at[pl.ds(cid * N_NODES + (NS - 1) * OUT_ROWS_PT,
                                       last)])

        cw = N_EDGES // NS
        for k in range(5):
            piece = pl.ds(0, cw // 5)
            csl = pl.ds(sid * cw + k * (cw // 5), cw // 5)
            pltpu.sync_copy(c_s.at[csl], zbuf_v.at[piece])

            @pl.when(cid == 0)
            def _():
                pltpu.sync_copy(zbuf_v.at[piece], out_c0_hbm.at[csl])

            @pl.when(cid == 1)
            def _():
                pltpu.sync_copy(zbuf_v.at[piece], out_c1_hbm.at[csl])

    return sc_kernel(src2, dst_p, rel_p, nodes_split, z2d, z1d)


BLK = 400
GRID = N_NODES // BLK
RIGHT_OFF = N_NODES // BLK


def _tc_finish(x, node_part, C0, C1, Ws, bs2, Wm, bm2, rel_emb, gamma2, beta2):
    prec = lax.Precision.DEFAULT

    def body(x_ref, l_ref, r_ref, c0_ref, c1_ref, ws_ref, bs_ref, wm_ref,
             bm_ref, rel_ref, g_ref, b_ref, o_ref):
        xb = x_ref[...]
        cb = c0_ref[...] + c1_ref[...]
        deg = jnp.sum(cb, axis=1, keepdims=True)
        mrel = jnp.dot(rel_ref[...], wm_ref[...],
                       preferred_element_type=jnp.float32, precision=prec)
        t = (jnp.dot(l_ref[...], wm_ref[:HALF, :],
                     preferred_element_type=jnp.float32, precision=prec)
             + jnp.dot(r_ref[...], wm_ref[HALF:, :],
                       preferred_element_type=jnp.float32, precision=prec)
             + jnp.dot(cb, mrel, preferred_element_type=jnp.float32,
                       precision=prec))
        agg = (t + deg * bm_ref[...]) / jnp.maximum(deg, 1.0)
        u = (jnp.dot(xb, ws_ref[...],
                     preferred_element_type=jnp.float32, precision=prec)
             + bs_ref[...] + agg)
        g = 0.5 * u * (1.0 + lax.erf(u * 0.7071067811865476))
        r = g + xb
        mu = jnp.mean(r, axis=1, keepdims=True)
        var = jnp.mean((r - mu) ** 2, axis=1, keepdims=True)
        o_ref[...] = (r - mu) * lax.rsqrt(var + 1e-5) * g_ref[...] + b_ref[...]

    full = lambda shape: pl.BlockSpec(shape, lambda i: (0, 0))
    return pl.pallas_call(
        body,
        grid=(GRID,),
        in_specs=[
            pl.BlockSpec((BLK, HIDDEN), lambda i: (i, 0)),
            pl.BlockSpec((BLK, HALF), lambda i: (i, 0)),
            pl.BlockSpec((BLK, HALF), lambda i: (i + RIGHT_OFF, 0)),
            pl.BlockSpec((BLK, NUM_REL), lambda i: (i, 0)),
            pl.BlockSpec((BLK, NUM_REL), lambda i: (i, 0)),
            full((HIDDEN, HIDDEN)),
            full((1, HIDDEN)),
            full((HIDDEN, HIDDEN)),
            full((1, HIDDEN)),
            full((NUM_REL, HIDDEN)),
            full((1, HIDDEN)),
            full((1, HIDDEN)),
        ],
        out_specs=pl.BlockSpec((BLK, HIDDEN), lambda i: (i, 0)),
        out_shape=jax.ShapeDtypeStruct((N_NODES, HIDDEN), jnp.float32),
    )(x, node_part, node_part, C0, C1, Ws, bs2, Wm, bm2, rel_emb, gamma2,
      beta2)


def kernel(node_states, Ws, bs, Wm, bm, rel_emb, ln_gamma, ln_beta,
           edge_index, edge_type_ids):
    src = edge_index[0].astype(jnp.int32)
    dst = edge_index[1].astype(jnp.int32)
    rel = jnp.clip(edge_type_ids.astype(jnp.int32), 0, NUM_REL - 1)

    pad = EPT_PAD - EPT
    src_p = jnp.pad(src.reshape(NS, EPT), ((0, 0), (0, pad))).reshape(-1)
    dst_p = jnp.pad(dst.reshape(NS, EPT), ((0, 0), (0, pad)),
                    constant_values=N_NODES).reshape(-1)
    rel_p = jnp.pad(rel.reshape(NS, EPT), ((0, 0), (0, pad))).reshape(-1)
    src2 = jnp.concatenate([2 * src_p, 2 * src_p + 1])
    nodes_split = node_states.reshape(2 * N_NODES, HALF)
    z2d = jnp.zeros((ZROWS, HALF), jnp.float32)
    z1d = jnp.zeros((ZBUF,), jnp.float32)

    node_part, c0, c1 = _sc_aggregate(src2, dst_p, rel_p, nodes_split,
                                      z2d, z1d)
    C0 = c0.reshape(N_NODES, NUM_REL)
    C1 = c1.reshape(N_NODES, NUM_REL)

    return _tc_finish(node_states, node_part, C0, C1, Ws,
                      bs.reshape(1, HIDDEN), Wm, bm.reshape(1, HIDDEN),
                      rel_emb, ln_gamma.reshape(1, HIDDEN),
                      ln_beta.reshape(1, HIDDEN))
